# Initial kernel scaffold; baseline (speedup 1.0000x reference)
import jax
import jax.numpy as jnp
from jax.experimental import pallas as pl


def kernel(prop_types, hut_colors, hut_rotations, tree_types, plant_types, windmill_rotations, tower_rotations, tent_rotations, terrain, table_0, table_1, table_2, table_3, table_4, table_5, table_6, table_7, table_8):
    raise NotImplementedError("write your pallas kernel here")


# speedup vs baseline: 6.6414x; 6.6414x over previous
import functools

import jax
import jax.numpy as jnp
from jax import lax
from jax.experimental import pallas as pl
from jax.experimental.pallas import tpu as pltpu
from jax.experimental.pallas import tpu_sc as plsc

NC = 2
NS = 16
NW = NC * NS
L = 16

B = 1024
W = 25
D = 25
WD = W * D
E = 128
V = 16
NE = 9
R = NE * V
NG = 40
PWD = NG * L
ACC = E * WD + L


def _body(colT_hbm, idx_hbm, out_hbm, colT_v, idx_v, acc_v):
    wid = lax.axis_index("s") * NC + lax.axis_index("c")
    pltpu.sync_copy(colT_hbm, colT_v)

    def task_body(t, carry):
        b = t * NW + wid
        pltpu.sync_copy(idx_hbm.at[b], idx_v)

        def group_body(gi, c):
            g = lax.rem(gi + NG - 1, NG)
            col = g * L
            ivs = [idx_v[i, pl.ds(col, L)] + jnp.int32(i * V)
                   for i in range(NE)]

            def e_body(e, cc):
                base = e * R
                s = plsc.load_gather(colT_v, [ivs[0] + base])
                for i in range(1, NE):
                    s = s + plsc.load_gather(colT_v, [ivs[i] + base])
                acc_v[pl.ds(e * WD + col, L)] = s
                return cc

            lax.fori_loop(0, E, e_body, c, unroll=8)
            return c

        lax.fori_loop(0, NG, group_body, 0)
        pltpu.sync_copy(acc_v.at[pl.ds(0, E * WD)], out_hbm.at[b])
        return carry

    lax.fori_loop(0, B // NW, task_body, 0)


@jax.jit
def _embed(colT, idx9):
    mesh = plsc.VectorSubcoreMesh(core_axis_name="c", subcore_axis_name="s")
    f = pl.kernel(
        _body,
        out_type=jax.ShapeDtypeStruct((B, E * WD), jnp.float32),
        mesh=mesh,
        scratch_types=[
            pltpu.VMEM((E * R,), jnp.float32),
            pltpu.VMEM((NE, PWD), jnp.int32),
            pltpu.VMEM((ACC,), jnp.float32),
        ],
        compiler_params=pltpu.CompilerParams(needs_layout_passes=False),
    )
    return f(colT, idx9)


def kernel(prop_types, hut_colors, hut_rotations, tree_types, plant_types,
           windmill_rotations, tower_rotations, tent_rotations, terrain,
           table_0, table_1, table_2, table_3, table_4, table_5, table_6,
           table_7, table_8):
    idxs = [prop_types, hut_colors, hut_rotations, tree_types, plant_types,
            windmill_rotations, tower_rotations, tent_rotations, terrain]
    tables = [table_0, table_1, table_2, table_3, table_4, table_5, table_6,
              table_7, table_8]

    idx9 = jnp.stack(
        [a.reshape(B, WD).astype(jnp.int32) for a in idxs], axis=1)
    idx9 = jnp.pad(idx9, ((0, 0), (0, 0), (0, PWD - WD)))

    tbl = jnp.stack([t.astype(jnp.float32) for t in tables])
    tbl = tbl.at[:8, 0, :].set(0.0)
    colT = tbl.reshape(R, E).T.reshape(-1)

    out = _embed(colT, idx9)
    return out.reshape(B, E, W, D)


# speedup vs baseline: 7.1700x; 1.0796x over previous
import functools

import jax
import jax.numpy as jnp
from jax import lax
from jax.experimental import pallas as pl
from jax.experimental.pallas import tpu as pltpu
from jax.experimental.pallas import tpu_sc as plsc

NC = 2
NS = 16
NW = NC * NS
L = 16

B = 1024
W = 25
D = 25
WD = W * D
E = 128
V = 16
NE = 9
R = NE * V
NG = 40
PWD = NG * L
ACC = E * WD + L


def _body(colT_hbm, idx_hbm, out_hbm, colT_v, idx_v, acc_v):
    wid = lax.axis_index("s") * NC + lax.axis_index("c")
    pltpu.sync_copy(colT_hbm, colT_v)

    def task_body(t, carry):
        b = t * NW + wid
        pltpu.sync_copy(idx_hbm.at[b], idx_v)

        def group_body(gi, c):
            g = lax.rem(gi + NG - 1, NG)
            col = g * L
            ivs = [idx_v[i, pl.ds(col, L)] + jnp.int32(i * V)
                   for i in range(NE)]

            def e_body(e, cc):
                tref = colT_v.at[pl.ds(e * R, R)]
                g = [plsc.load_gather(tref, [ivs[i]]) for i in range(NE)]
                s = (((g[0] + g[1]) + (g[2] + g[3]))
                     + ((g[4] + g[5]) + (g[6] + g[7])) + g[8])
                acc_v[pl.ds(e * WD + col, L)] = s
                return cc

            lax.fori_loop(0, E, e_body, c, unroll=8)
            return c

        lax.fori_loop(0, NG, group_body, 0)
        pltpu.sync_copy(acc_v.at[pl.ds(0, E * WD)], out_hbm.at[b])
        return carry

    lax.fori_loop(0, B // NW, task_body, 0)


@jax.jit
def _embed(colT, idx9):
    mesh = plsc.VectorSubcoreMesh(core_axis_name="c", subcore_axis_name="s")
    f = pl.kernel(
        _body,
        out_type=jax.ShapeDtypeStruct((B, E * WD), jnp.float32),
        mesh=mesh,
        scratch_types=[
            pltpu.VMEM((E * R,), jnp.float32),
            pltpu.VMEM((NE, PWD), jnp.int32),
            pltpu.VMEM((ACC,), jnp.float32),
        ],
        compiler_params=pltpu.CompilerParams(needs_layout_passes=False),
    )
    return f(colT, idx9)


def kernel(prop_types, hut_colors, hut_rotations, tree_types, plant_types,
           windmill_rotations, tower_rotations, tent_rotations, terrain,
           table_0, table_1, table_2, table_3, table_4, table_5, table_6,
           table_7, table_8):
    idxs = [prop_types, hut_colors, hut_rotations, tree_types, plant_types,
            windmill_rotations, tower_rotations, tent_rotations, terrain]
    tables = [table_0, table_1, table_2, table_3, table_4, table_5, table_6,
              table_7, table_8]

    idx9 = jnp.stack(
        [a.reshape(B, WD).astype(jnp.int32) for a in idxs], axis=1)
    idx9 = jnp.pad(idx9, ((0, 0), (0, 0), (0, PWD - WD)))

    tbl = jnp.stack([t.astype(jnp.float32) for t in tables])
    tbl = tbl.at[:8, 0, :].set(0.0)
    colT = tbl.reshape(R, E).T.reshape(-1)

    out = _embed(colT, idx9)
    return out.reshape(B, E, W, D)


# speedup vs baseline: 11.2459x; 1.5685x over previous
import functools

import jax
import jax.numpy as jnp
from jax import lax
from jax.experimental import pallas as pl
from jax.experimental.pallas import tpu as pltpu
from jax.experimental.pallas import tpu_sc as plsc

NC = 2
NS = 16
NW = NC * NS
L = 16

B = 1024
W = 25
D = 25
WD = W * D
E = 128
V = 16
NE = 9
R = NE * V
NG = 40
GB = 4
PWD = NG * L
ACC = E * WD + L


def _body(colT_hbm, idx_hbm, out_hbm, colT_v, idx_v, acc_v):
    wid = lax.axis_index("s") * NC + lax.axis_index("c")
    pltpu.sync_copy(colT_hbm, colT_v)

    dnums = lax.GatherDimensionNumbers(
        offset_dims=(), collapsed_slice_dims=(0,), start_index_map=(0,))

    def vgather(col, iv):
        return lax.gather(col, iv[:, None], dnums, (1,),
                          mode=lax.GatherScatterMode.PROMISE_IN_BOUNDS)

    def task_body(t, carry):
        b = t * NW + wid
        pltpu.sync_copy(idx_hbm.at[b], idx_v)

        for gbase in [NG - GB] + list(range(0, NG - GB, GB)):
            ivs = [[idx_v[i, pl.ds((gbase + g) * L, L)] for i in range(NE)]
                   for g in range(GB)]

            def e_body(e, cc, ivs=ivs, gbase=gbase):
                cols = [colT_v[pl.ds(e * R + i * V, V)] for i in range(NE)]
                for g in range(GB):
                    gv = [vgather(cols[i], ivs[g][i]) for i in range(NE)]
                    s = (((gv[0] + gv[1]) + (gv[2] + gv[3]))
                         + ((gv[4] + gv[5]) + (gv[6] + gv[7])) + gv[8])
                    acc_v[pl.ds(e * WD + (gbase + g) * L, L)] = s
                return cc

            lax.fori_loop(0, E, e_body, 0, unroll=2)

        pltpu.sync_copy(acc_v.at[pl.ds(0, E * WD)], out_hbm.at[b])
        return carry

    lax.fori_loop(0, B // NW, task_body, 0)


@jax.jit
def _embed(colT, idx9):
    mesh = plsc.VectorSubcoreMesh(core_axis_name="c", subcore_axis_name="s")
    f = pl.kernel(
        _body,
        out_type=jax.ShapeDtypeStruct((B, E * WD), jnp.float32),
        mesh=mesh,
        scratch_types=[
            pltpu.VMEM((E * R,), jnp.float32),
            pltpu.VMEM((NE, PWD), jnp.int32),
            pltpu.VMEM((ACC,), jnp.float32),
        ],
        compiler_params=pltpu.CompilerParams(needs_layout_passes=False),
    )
    return f(colT, idx9)


def kernel(prop_types, hut_colors, hut_rotations, tree_types, plant_types,
           windmill_rotations, tower_rotations, tent_rotations, terrain,
           table_0, table_1, table_2, table_3, table_4, table_5, table_6,
           table_7, table_8):
    idxs = [prop_types, hut_colors, hut_rotations, tree_types, plant_types,
            windmill_rotations, tower_rotations, tent_rotations, terrain]
    tables = [table_0, table_1, table_2, table_3, table_4, table_5, table_6,
              table_7, table_8]

    idx9 = jnp.stack(
        [a.reshape(B, WD).astype(jnp.int32) for a in idxs], axis=1)
    idx9 = jnp.pad(idx9, ((0, 0), (0, 0), (0, PWD - WD)))

    tbl = jnp.stack([t.astype(jnp.float32) for t in tables])
    tbl = tbl.at[:8, 0, :].set(0.0)
    colT = tbl.reshape(R, E).T.reshape(-1)

    out = _embed(colT, idx9)
    return out.reshape(B, E, W, D)
